# baseline (speedup 1.0000x reference)
import math

import jax
import jax.numpy as jnp
from jax import lax
from jax.experimental import pallas as pl
from jax.experimental.pallas import tpu as pltpu
from jax.experimental.pallas import tpu_sc as plsc

_NC, _NS, _L = 2, 16, 16
_NW = _NC * _NS

_B, _S, _D = 4, 2048, 1024
_NTOK = _B * _S
_TPW = _NTOK // _NW
_CHUNK = 32
_NCHUNK = _TPW // _CHUNK
_SCALE = math.sqrt(_D)


def _pos_encoding(seq_len, d_model):
    position = jnp.arange(seq_len, dtype=jnp.float32)[:, None]
    div_term = jnp.exp(
        jnp.arange(0, d_model, 2, dtype=jnp.float32)
        * (-math.log(10000.0) / d_model))
    pe = jnp.zeros((seq_len, d_model), dtype=jnp.float32)
    pe = pe.at[:, 0::2].set(jnp.sin(position * div_term))
    pe = pe.at[:, 1::2].set(jnp.cos(position * div_term))
    return pe


@jax.jit
def _embed(idx_flat, table, pe):
    mesh = plsc.VectorSubcoreMesh(
        core_axis_name="c", subcore_axis_name="s",
        num_cores=_NC, num_subcores=_NS)

    @pl.kernel(
        out_type=jax.ShapeDtypeStruct((_NTOK, _D), jnp.float32),
        mesh=mesh,
        scratch_types=[
            pltpu.VMEM((_TPW,), jnp.int32),
            pltpu.VMEM((_CHUNK, _D), jnp.float32),
            pltpu.VMEM((_CHUNK, _D), jnp.float32),
            pltpu.SemaphoreType.DMA,
        ],
    )
    def body(idx_hbm, table_hbm, pe_hbm, out_hbm, idx_v, rows_v, pe_v, sem):
        wid = lax.axis_index("s") * _NC + lax.axis_index("c")
        tok_base = wid * _TPW
        pos_base = lax.rem(tok_base, _S)
        pltpu.sync_copy(idx_hbm.at[pl.ds(tok_base, _TPW)], idx_v)
        for c in range(_NCHUNK):
            gat = pltpu.async_copy(
                table_hbm.at[idx_v.at[pl.ds(c * _CHUNK, _CHUNK)]], rows_v, sem)
            pltpu.sync_copy(
                pe_hbm.at[pl.ds(pos_base + c * _CHUNK, _CHUNK)], pe_v)
            gat.wait()

            def fma(r, _):
                for j in range(_D // _L):
                    sl = pl.ds(j * _L, _L)
                    rows_v[r, sl] = rows_v[r, sl] * _SCALE + pe_v[r, sl]
                return _

            lax.fori_loop(0, _CHUNK, fma, 0)
            pltpu.sync_copy(
                rows_v, out_hbm.at[pl.ds(tok_base + c * _CHUNK, _CHUNK)])

    return body(idx_flat, table, pe)


def kernel(x, table):
    pe = _pos_encoding(_S, _D)
    idx_flat = x.astype(jnp.int32).reshape(_NTOK)
    out = _embed(idx_flat, table, pe)
    return out.reshape(_B, _S, _D)


# speedup vs baseline: 1.2022x; 1.2022x over previous
import math

import jax
import jax.numpy as jnp
from jax import lax
from jax.experimental import pallas as pl
from jax.experimental.pallas import tpu as pltpu
from jax.experimental.pallas import tpu_sc as plsc

_NC, _NS, _L = 2, 16, 16
_NW = _NC * _NS

_B, _S, _D = 4, 2048, 1024
_NTOK = _B * _S
_CHUNK = 32
_NCHUNK = 8
_SCALE = math.sqrt(_D)


def _pos_encoding(seq_len, d_model):
    position = jnp.arange(seq_len, dtype=jnp.float32)[:, None]
    div_term = jnp.exp(
        jnp.arange(0, d_model, 2, dtype=jnp.float32)
        * (-math.log(10000.0) / d_model))
    pe = jnp.zeros((seq_len, d_model), dtype=jnp.float32)
    pe = pe.at[:, 0::2].set(jnp.sin(position * div_term))
    pe = pe.at[:, 1::2].set(jnp.cos(position * div_term))
    return pe


@jax.jit
def _embed(idx_arr, table, pe):
    mesh = plsc.VectorSubcoreMesh(
        core_axis_name="c", subcore_axis_name="s",
        num_cores=_NC, num_subcores=_NS)

    @pl.kernel(
        out_type=jax.ShapeDtypeStruct((_NTOK, _D), jnp.float32),
        mesh=mesh,
        scratch_types=[
            pltpu.VMEM((_NCHUNK * _CHUNK,), jnp.int32),
            pltpu.VMEM((_CHUNK, _D), jnp.float32),
            pltpu.VMEM((_CHUNK, _D), jnp.float32),
            pltpu.VMEM((_CHUNK, _D), jnp.float32),
            pltpu.SemaphoreType.DMA,
            pltpu.SemaphoreType.DMA,
            pltpu.SemaphoreType.DMA,
            pltpu.SemaphoreType.DMA,
            pltpu.SemaphoreType.DMA,
        ],
    )
    def body(idx_hbm, table_hbm, pe_hbm, out_hbm,
             idx_v, rows0, rows1, pe_v, g0, g1, s0, s1, psem):
        wid = lax.axis_index("s") * _NC + lax.axis_index("c")
        rows = (rows0, rows1)
        gsem = (g0, g1)
        ssem = (s0, s1)
        pltpu.sync_copy(idx_hbm.at[wid], idx_v)

        def gather(k, buf):
            return pltpu.async_copy(
                table_hbm.at[idx_v.at[pl.ds(k * _CHUNK, _CHUNK)]],
                rows[buf], gsem[buf])

        def fma(buf):
            def row_fma(r, carry):
                for j in range(_D // _L):
                    sl = pl.ds(j * _L, _L)
                    rows[buf][r, sl] = rows[buf][r, sl] * _SCALE + pe_v[r, sl]
                return carry
            lax.fori_loop(0, _CHUNK, row_fma, 0)

        pe_d = pltpu.async_copy(
            pe_hbm.at[pl.ds(_CHUNK * wid, _CHUNK)], pe_v, psem)
        g_d = [gather(0, 0), None]
        s_d = [None, None]
        for k in range(_NCHUNK):
            outer, batch = divmod(k, 4)
            buf = k & 1
            if k + 1 < _NCHUNK:
                nbuf = (k + 1) & 1
                if s_d[nbuf] is not None:
                    s_d[nbuf].wait()
                g_d[nbuf] = gather(k + 1, nbuf)
            g_d[buf].wait()
            if k == 0 or k == 4:
                pe_d.wait()
            fma(buf)
            out_base = batch * _S + outer * (_S // 2) + _CHUNK * wid
            s_d[buf] = pltpu.async_copy(
                rows[buf], out_hbm.at[pl.ds(out_base, _CHUNK)], ssem[buf])
            if k == 3:
                pe_d = pltpu.async_copy(
                    pe_hbm.at[pl.ds(_S // 2 + _CHUNK * wid, _CHUNK)],
                    pe_v, psem)
        s_d[0].wait()
        s_d[1].wait()

    return body(idx_arr, table, pe)


def kernel(x, table):
    pe = _pos_encoding(_S, _D)
    idx_arr = (x.astype(jnp.int32)
               .reshape(_B, 2, _NW, _CHUNK)
               .transpose(2, 1, 0, 3)
               .reshape(_NW, _NCHUNK * _CHUNK))
    out = _embed(idx_arr, table, pe)
    return out.reshape(_B, _S, _D)


# speedup vs baseline: 1.9150x; 1.5929x over previous
import math

import jax
import jax.numpy as jnp
import numpy as np
from jax import lax
from jax.experimental import pallas as pl
from jax.experimental.pallas import tpu as pltpu
from jax.experimental.pallas import tpu_sc as plsc

_NC, _NS, _L = 2, 16, 16
_NW = _NC * _NS

_B, _S, _D = 4, 2048, 1024
_NTOK = _B * _S
_CHUNK = 32
_NCHUNK = 8
_SCALE = math.sqrt(_D)


def _pos_encoding(seq_len, d_model):
    position = np.arange(seq_len, dtype=np.float32)[:, None]
    div_term = np.exp(
        np.arange(0, d_model, 2, dtype=np.float32)
        * (-math.log(10000.0) / d_model))
    pe = np.zeros((seq_len, d_model), dtype=np.float32)
    pe[:, 0::2] = np.sin(position * div_term)
    pe[:, 1::2] = np.cos(position * div_term)
    return pe


_PE = _pos_encoding(_S, _D)


@jax.jit
def _embed(idx_arr, table, pe):
    mesh = plsc.VectorSubcoreMesh(
        core_axis_name="c", subcore_axis_name="s",
        num_cores=_NC, num_subcores=_NS)

    @pl.kernel(
        out_type=jax.ShapeDtypeStruct((_NTOK, _D), jnp.float32),
        mesh=mesh,
        scratch_types=[
            pltpu.VMEM((_NCHUNK * _CHUNK,), jnp.int32),
            pltpu.VMEM((_CHUNK, _D), jnp.float32),
            pltpu.VMEM((_CHUNK, _D), jnp.float32),
            pltpu.VMEM((_CHUNK, _D), jnp.float32),
            pltpu.SemaphoreType.DMA,
            pltpu.SemaphoreType.DMA,
            pltpu.SemaphoreType.DMA,
            pltpu.SemaphoreType.DMA,
            pltpu.SemaphoreType.DMA,
        ],
    )
    def body(idx_hbm, table_hbm, pe_hbm, out_hbm,
             idx_v, rows0, rows1, pe_v, g0, g1, s0, s1, psem):
        wid = lax.axis_index("s") * _NC + lax.axis_index("c")
        rows = (rows0, rows1)
        gsem = (g0, g1)
        ssem = (s0, s1)
        pltpu.sync_copy(idx_hbm.at[wid], idx_v)

        def gather(k, buf):
            return pltpu.async_copy(
                table_hbm.at[idx_v.at[pl.ds(k * _CHUNK, _CHUNK)]],
                rows[buf], gsem[buf])

        def fma(buf):
            def row_fma(r, carry):
                for j in range(_D // _L):
                    sl = pl.ds(j * _L, _L)
                    rows[buf][r, sl] = rows[buf][r, sl] * _SCALE + pe_v[r, sl]
                return carry
            lax.fori_loop(0, _CHUNK, row_fma, 0)

        pe_d = pltpu.async_copy(
            pe_hbm.at[pl.ds(_CHUNK * wid, _CHUNK)], pe_v, psem)
        g_d = [gather(0, 0), None]
        s_d = [None, None]
        for k in range(_NCHUNK):
            outer, batch = divmod(k, 4)
            buf = k & 1
            if k + 1 < _NCHUNK:
                nbuf = (k + 1) & 1
                if s_d[nbuf] is not None:
                    s_d[nbuf].wait()
                g_d[nbuf] = gather(k + 1, nbuf)
            g_d[buf].wait()
            if k == 0 or k == 4:
                pe_d.wait()
            fma(buf)
            out_base = batch * _S + outer * (_S // 2) + _CHUNK * wid
            s_d[buf] = pltpu.async_copy(
                rows[buf], out_hbm.at[pl.ds(out_base, _CHUNK)], ssem[buf])
            if k == 3:
                pe_d = pltpu.async_copy(
                    pe_hbm.at[pl.ds(_S // 2 + _CHUNK * wid, _CHUNK)],
                    pe_v, psem)
        s_d[0].wait()
        s_d[1].wait()

    return body(idx_arr, table, pe)


def kernel(x, table):
    pe = jnp.asarray(_PE)
    idx_arr = (x.astype(jnp.int32)
               .reshape(_B, 2, _NW, _CHUNK)
               .transpose(2, 1, 0, 3)
               .reshape(_NW, _NCHUNK * _CHUNK))
    out = _embed(idx_arr, table, pe)
    return out.reshape(_B, _S, _D)
